# Initial kernel scaffold; baseline (speedup 1.0000x reference)
import jax
import jax.numpy as jnp
from jax.experimental import pallas as pl


def kernel(features, edge, edge_weight, W1, b1, W2, b2, fc_W, fc_b):
    raise NotImplementedError("write your pallas kernel here")


# speedup vs baseline: 4.0436x; 4.0436x over previous
import functools

import jax
import jax.numpy as jnp
from jax import lax
from jax.experimental import pallas as pl
from jax.experimental.pallas import tpu as pltpu
from jax.experimental.pallas import tpu_sc as plsc

N_NODES = 10000
N_EDGES = 320000
D = 128

NC = 2
NS = 16
NW = NC * NS
E_PER_W = N_EDGES // NW
C = 80
N_CHUNKS = E_PER_W // C
ROWS_PER_TILE = 632
NP = ROWS_PER_TILE * NS
ZB = 158

def _spmm_body(xw_hbm, row_hbm, col_hbm, w_hbm, out_hbm,
               col_v, row_v, w_v, rows_v, zero_v, acc_sh, gsem):
    c = lax.axis_index("c")
    s = lax.axis_index("s")
    wid = s * NC + c

    def _zfill(r, _):
        for jj in range(D // 16):
            zero_v[r, pl.ds(jj * 16, 16)] = jnp.zeros((16,), jnp.float32)
        return 0
    lax.fori_loop(0, ZB, _zfill, 0)
    for j in range(ROWS_PER_TILE // ZB):
        pltpu.sync_copy(zero_v, acc_sh.at[pl.ds(s * ROWS_PER_TILE + j * ZB, ZB)])
    plsc.subcore_barrier()

    def _chunk(i, _):
        base = wid * E_PER_W + i * C
        pltpu.sync_copy(col_hbm.at[pl.ds(base, C)], col_v)
        pltpu.sync_copy(row_hbm.at[pl.ds(base, C)], row_v)
        pltpu.sync_copy(w_hbm.at[pl.ds(base, C)], w_v)
        pltpu.async_copy(xw_hbm.at[col_v], rows_v, gsem).wait()

        def _scale(g, _):
            wv = w_v[pl.ds(g * 16, 16)]
            for k in range(16):
                e = g * 16 + k
                we = wv[k]
                for jj in range(D // 16):
                    sl = pl.ds(jj * 16, 16)
                    rows_v[e, sl] = rows_v[e, sl] * we
            return 0
        lax.fori_loop(0, C // 16, _scale, 0)
        pltpu.sync_copy(rows_v, acc_sh.at[row_v], add=True)
        return 0
    lax.fori_loop(0, N_CHUNKS, _chunk, 0)
    plsc.subcore_barrier()

    pltpu.sync_copy(acc_sh.at[pl.ds(s * ROWS_PER_TILE, ROWS_PER_TILE)],
                    out_hbm.at[c, pl.ds(s * ROWS_PER_TILE, ROWS_PER_TILE)])


@functools.cache
def _make_spmm_sc():
    mesh = plsc.VectorSubcoreMesh(core_axis_name="c", subcore_axis_name="s")
    return pl.kernel(
        _spmm_body,
        out_type=jax.ShapeDtypeStruct((NC, NP, D), jnp.float32),
        mesh=mesh,
        scratch_types=[
            pltpu.VMEM((C,), jnp.int32),
            pltpu.VMEM((C,), jnp.int32),
            pltpu.VMEM((C,), jnp.float32),
            pltpu.VMEM((C, D), jnp.float32),
            pltpu.VMEM((ZB, D), jnp.float32),
            pltpu.VMEM_SHARED((NP, D), jnp.float32),
            pltpu.SemaphoreType.DMA,
        ],
        name="spmm_sc",
    )


_BLK = 400
_GRID = N_NODES // _BLK


def _mm_body(x_ref, w_ref, o_ref):
    o_ref[...] = jnp.dot(x_ref[...], w_ref[...],
                         preferred_element_type=jnp.float32)


_mm_tc = pl.pallas_call(
    _mm_body,
    grid=(_GRID,),
    in_specs=[
        pl.BlockSpec((_BLK, D), lambda i: (i, 0)),
        pl.BlockSpec((D, D), lambda i: (0, 0)),
    ],
    out_specs=pl.BlockSpec((_BLK, D), lambda i: (i, 0)),
    out_shape=jax.ShapeDtypeStruct((N_NODES, D), jnp.float32),
)


def _fuse_body(p0_ref, p1_ref, b_ref, w_ref, bo_ref, o_ref):
    h = jnp.tanh(p0_ref[...] + p1_ref[...] + b_ref[...])
    o_ref[...] = jnp.dot(h, w_ref[...],
                         preferred_element_type=jnp.float32) + bo_ref[...]


_BLKF = ROWS_PER_TILE
_GRIDF = NP // _BLKF

_fuse_tc = pl.pallas_call(
    _fuse_body,
    grid=(_GRIDF,),
    in_specs=[
        pl.BlockSpec((_BLKF, D), lambda i: (i, 0)),
        pl.BlockSpec((_BLKF, D), lambda i: (i, 0)),
        pl.BlockSpec((1, D), lambda i: (0, 0)),
        pl.BlockSpec((D, D), lambda i: (0, 0)),
        pl.BlockSpec((1, D), lambda i: (0, 0)),
    ],
    out_specs=pl.BlockSpec((_BLKF, D), lambda i: (i, 0)),
    out_shape=jax.ShapeDtypeStruct((NP, D), jnp.float32),
)


def kernel(features, edge, edge_weight, W1, b1, W2, b2, fc_W, fc_b):
    row = edge[0].astype(jnp.int32)
    col = edge[1].astype(jnp.int32)
    zeros_row = jnp.zeros((1, D), jnp.float32)

    spmm = _make_spmm_sc()
    xw1 = _mm_tc(features, W1)
    p = spmm(xw1, row, col, edge_weight)
    xw2 = _fuse_tc(p[0], p[1], b1.reshape(1, D), W2, zeros_row)
    p2 = spmm(xw2, row, col, edge_weight)
    fcW_pad = jnp.pad(fc_W, ((0, 0), (0, D - fc_W.shape[1])))
    fcb_pad = jnp.pad(fc_b, (0, D - fc_b.shape[0])).reshape(1, D)
    out_full = _fuse_tc(p2[0], p2[1], b2.reshape(1, D), fcW_pad, fcb_pad)
    return out_full[:N_NODES, :fc_W.shape[1]]


# speedup vs baseline: 5.3517x; 1.3235x over previous
import functools

import jax
import jax.numpy as jnp
from jax import lax
from jax.experimental import pallas as pl
from jax.experimental.pallas import tpu as pltpu
from jax.experimental.pallas import tpu_sc as plsc

N_NODES = 10000
N_EDGES = 320000
D = 128

NC = 2
NS = 16
NW = NC * NS
C = 128
N_CHUNKS = 79
E_PER_W = N_CHUNKS * C
NE_PAD = E_PER_W * NW
ROWS_PER_TILE = 632
NP = ROWS_PER_TILE * NS
ZB = 79

NBUF = 2
MBUF = 4


def _spmm_body(xw_hbm, idx_hbm, w_hbm, out_hbm,
               rb0, rb1, mb0, mb1, mb2, mb3, wb0, wb1, wb2, wb3, acc_sh,
               g0, g1, s0, s1, m0, m1, m2, m3):
    rb = [rb0, rb1]
    mb = [mb0, mb1, mb2, mb3]
    wb = [wb0, wb1, wb2, wb3]
    gsem = [g0, g1]
    ssem = [s0, s1]
    msem = [m0, m1, m2, m3]
    c = lax.axis_index("c")
    s = lax.axis_index("s")
    wid = s * NC + c

    def _zfill(r, _):
        for jj in range(D // 16):
            rb0[r, pl.ds(jj * 16, 16)] = jnp.zeros((16,), jnp.float32)
        return 0
    lax.fori_loop(0, C, _zfill, 0)
    base_r = s * ROWS_PER_TILE
    for j in range(ROWS_PER_TILE // C):
        pltpu.sync_copy(rb0, acc_sh.at[pl.ds(base_r + j * C, C)])
    rem = ROWS_PER_TILE % C
    if rem:
        pltpu.sync_copy(rb0.at[pl.ds(0, rem)],
                        acc_sh.at[pl.ds(base_r + (ROWS_PER_TILE // C) * C, rem)])
    plsc.subcore_barrier()

    def _meta_start(g, m):
        pltpu.async_copy(idx_hbm.at[wid, g], mb[m], msem[m])
        pltpu.async_copy(w_hbm.at[wid, g], wb[m], msem[m])

    def _meta_wait(m):
        pltpu.make_async_copy(idx_hbm.at[wid, 0], mb[m], msem[m]).wait()
        pltpu.make_async_copy(w_hbm.at[wid, 0], wb[m], msem[m]).wait()

    def _gather_wait(b):
        pltpu.make_async_copy(xw_hbm.at[pl.ds(0, C)], rb[b], gsem[b]).wait()

    def _scatter_wait(b):
        pltpu.make_async_copy(rb[b], acc_sh.at[pl.ds(0, C)], ssem[b]).wait()

    _meta_start(0, 0)
    _meta_start(1, 1)
    _meta_wait(0)
    pltpu.async_copy(xw_hbm.at[mb[0].at[1]], rb[0], gsem[0])

    def _group(i, _):
        for bb in range(4):
            g = i * 4 + bb
            b = bb % NBUF
            m = bb

            @pl.when(g < N_CHUNKS)
            def _():
                _gather_wait(b)

                @pl.when(g >= 1)
                def _():
                    _scatter_wait(1 - b)

                @pl.when(g + 1 < N_CHUNKS)
                def _():
                    _meta_wait((m + 1) % MBUF)
                    pltpu.async_copy(xw_hbm.at[mb[(m + 1) % MBUF].at[1]],
                                     rb[1 - b], gsem[1 - b])

                @pl.when(g + 2 < N_CHUNKS)
                def _():
                    _meta_start(g + 2, (m + 2) % MBUF)

                def _scale(gg, _c):
                    wv = wb[m][pl.ds(gg * 16, 16)]
                    for k in range(16):
                        e = gg * 16 + k
                        we = wv[k]
                        for jj in range(D // 16):
                            sl = pl.ds(jj * 16, 16)
                            rb[b][e, sl] = rb[b][e, sl] * we
                    return 0
                lax.fori_loop(0, C // 16, _scale, 0)
                pltpu.async_copy(rb[b], acc_sh.at[mb[m].at[0]], ssem[b],
                                 add=True)
        return 0
    lax.fori_loop(0, (N_CHUNKS + 3) // 4, _group, 0)

    _scatter_wait((N_CHUNKS - 1) % NBUF)
    plsc.subcore_barrier()

    pltpu.sync_copy(acc_sh.at[pl.ds(s * ROWS_PER_TILE, ROWS_PER_TILE)],
                    out_hbm.at[c, pl.ds(s * ROWS_PER_TILE, ROWS_PER_TILE)])


@functools.cache
def _make_spmm_sc():
    mesh = plsc.VectorSubcoreMesh(core_axis_name="c", subcore_axis_name="s")
    return pl.kernel(
        _spmm_body,
        out_type=jax.ShapeDtypeStruct((NC, NP, D), jnp.float32),
        mesh=mesh,
        scratch_types=[
            pltpu.VMEM((C, D), jnp.float32),
            pltpu.VMEM((C, D), jnp.float32),
            pltpu.VMEM((2, C), jnp.int32),
            pltpu.VMEM((2, C), jnp.int32),
            pltpu.VMEM((2, C), jnp.int32),
            pltpu.VMEM((2, C), jnp.int32),
            pltpu.VMEM((C,), jnp.float32),
            pltpu.VMEM((C,), jnp.float32),
            pltpu.VMEM((C,), jnp.float32),
            pltpu.VMEM((C,), jnp.float32),
            pltpu.VMEM_SHARED((NP, D), jnp.float32),
            pltpu.SemaphoreType.DMA,
            pltpu.SemaphoreType.DMA,
            pltpu.SemaphoreType.DMA,
            pltpu.SemaphoreType.DMA,
            pltpu.SemaphoreType.DMA,
            pltpu.SemaphoreType.DMA,
            pltpu.SemaphoreType.DMA,
            pltpu.SemaphoreType.DMA,
        ],
        name="spmm_sc",
    )


_BLK = 400
_GRID = N_NODES // _BLK


def _mm_body(x_ref, w_ref, o_ref):
    o_ref[...] = jnp.dot(x_ref[...], w_ref[...],
                         preferred_element_type=jnp.float32)


_mm_tc = pl.pallas_call(
    _mm_body,
    grid=(_GRID,),
    in_specs=[
        pl.BlockSpec((_BLK, D), lambda i: (i, 0)),
        pl.BlockSpec((D, D), lambda i: (0, 0)),
    ],
    out_specs=pl.BlockSpec((_BLK, D), lambda i: (i, 0)),
    out_shape=jax.ShapeDtypeStruct((N_NODES, D), jnp.float32),
)


def _fuse_body(p0_ref, p1_ref, b_ref, w_ref, bo_ref, o_ref):
    h = jnp.tanh(p0_ref[...] + p1_ref[...] + b_ref[...])
    o_ref[...] = jnp.dot(h, w_ref[...],
                         preferred_element_type=jnp.float32) + bo_ref[...]


_BLKF = ROWS_PER_TILE
_GRIDF = NP // _BLKF

_fuse_tc = pl.pallas_call(
    _fuse_body,
    grid=(_GRIDF,),
    in_specs=[
        pl.BlockSpec((_BLKF, D), lambda i: (i, 0)),
        pl.BlockSpec((_BLKF, D), lambda i: (i, 0)),
        pl.BlockSpec((1, D), lambda i: (0, 0)),
        pl.BlockSpec((D, D), lambda i: (0, 0)),
        pl.BlockSpec((1, D), lambda i: (0, 0)),
    ],
    out_specs=pl.BlockSpec((_BLKF, D), lambda i: (i, 0)),
    out_shape=jax.ShapeDtypeStruct((NP, D), jnp.float32),
)


def kernel(features, edge, edge_weight, W1, b1, W2, b2, fc_W, fc_b):
    pad = NE_PAD - N_EDGES
    dst = jnp.pad(edge[0].astype(jnp.int32), (0, pad))
    srcx = jnp.pad(edge[1].astype(jnp.int32), (0, pad))
    idx = jnp.stack([dst, srcx])
    idx = idx.reshape(2, NW, N_CHUNKS, C).transpose(1, 2, 0, 3)
    w = jnp.pad(edge_weight, (0, pad)).reshape(NW, N_CHUNKS, C)
    zeros_row = jnp.zeros((1, D), jnp.float32)

    spmm = _make_spmm_sc()
    xw1 = _mm_tc(features, W1)
    p = spmm(xw1, idx, w)
    xw2 = _fuse_tc(p[0], p[1], b1.reshape(1, D), W2, zeros_row)
    p2 = spmm(xw2, idx, w)
    fcW_pad = jnp.pad(fc_W, ((0, 0), (0, D - fc_W.shape[1])))
    fcb_pad = jnp.pad(fc_b, (0, D - fc_b.shape[0])).reshape(1, D)
    out_full = _fuse_tc(p2[0], p2[1], b2.reshape(1, D), fcW_pad, fcb_pad)
    return out_full[:N_NODES, :fc_W.shape[1]]


# speedup vs baseline: 8.2375x; 1.5392x over previous
import functools

import jax
import jax.numpy as jnp
from jax import lax
from jax.experimental import pallas as pl
from jax.experimental.pallas import tpu as pltpu
from jax.experimental.pallas import tpu_sc as plsc

N_NODES = 10000
N_EDGES = 320000
D = 128

NC = 2
NS = 16
NW = NC * NS
C = 128
K0 = 52
K1 = 106
TOTAL_CHUNKS = NS * (K0 + K1)
NE_PAD = TOTAL_CHUNKS * C
ROWS_PER_TILE = 632
NP = ROWS_PER_TILE * NS
ZB = 79

NBUF = 2
MBUF = 4


def _spmm_body(xw_hbm, idx_hbm, w_hbm, out_hbm,
               rb0, rb1, mb0, mb1, mb2, mb3, wb0, wb1, wb2, wb3, acc_sh,
               g0, g1, s0, s1, m0, m1, m2, m3):
    rb = [rb0, rb1]
    mb = [mb0, mb1, mb2, mb3]
    wb = [wb0, wb1, wb2, wb3]
    gsem = [g0, g1]
    ssem = [s0, s1]
    msem = [m0, m1, m2, m3]
    c = lax.axis_index("c")
    s = lax.axis_index("s")
    n_chunks = jnp.where(c == 0, K0, K1)
    chunk0 = c * NS * K0 + s * n_chunks

    def _zfill(r, _):
        for jj in range(D // 16):
            rb0[r, pl.ds(jj * 16, 16)] = jnp.zeros((16,), jnp.float32)
        return 0
    lax.fori_loop(0, C, _zfill, 0)
    base_r = s * ROWS_PER_TILE
    for j in range(ROWS_PER_TILE // C):
        pltpu.sync_copy(rb0, acc_sh.at[pl.ds(base_r + j * C, C)])
    rem = ROWS_PER_TILE % C
    if rem:
        pltpu.sync_copy(rb0.at[pl.ds(0, rem)],
                        acc_sh.at[pl.ds(base_r + (ROWS_PER_TILE // C) * C, rem)])
    plsc.subcore_barrier()

    def _meta_start(g, m):
        pltpu.async_copy(idx_hbm.at[chunk0 + g], mb[m], msem[m])
        pltpu.async_copy(w_hbm.at[chunk0 + g], wb[m], msem[m])

    def _meta_wait(m):
        pltpu.make_async_copy(idx_hbm.at[0], mb[m], msem[m]).wait()
        pltpu.make_async_copy(w_hbm.at[0], wb[m], msem[m]).wait()

    def _gather_wait(b):
        pltpu.make_async_copy(xw_hbm.at[pl.ds(0, C)], rb[b], gsem[b]).wait()

    def _scatter_wait(b):
        pltpu.make_async_copy(rb[b], acc_sh.at[pl.ds(0, C)], ssem[b]).wait()

    _meta_start(0, 0)
    _meta_start(1, 1)
    _meta_wait(0)
    pltpu.async_copy(xw_hbm.at[mb[0].at[1]], rb[0], gsem[0])

    def _group(i, _):
        for bb in range(4):
            g = i * 4 + bb
            b = bb % NBUF
            m = bb

            @pl.when(g < n_chunks)
            def _():
                _gather_wait(b)

                @pl.when(g >= 1)
                def _():
                    _scatter_wait(1 - b)

                @pl.when(g + 1 < n_chunks)
                def _():
                    _meta_wait((m + 1) % MBUF)
                    pltpu.async_copy(xw_hbm.at[mb[(m + 1) % MBUF].at[1]],
                                     rb[1 - b], gsem[1 - b])

                @pl.when(g + 2 < n_chunks)
                def _():
                    _meta_start(g + 2, (m + 2) % MBUF)

                def _scale(gg, _c):
                    wv = wb[m][pl.ds(gg * 16, 16)]
                    for k in range(16):
                        e = gg * 16 + k
                        we = wv[k]
                        for jj in range(D // 16):
                            sl = pl.ds(jj * 16, 16)
                            rb[b][e, sl] = rb[b][e, sl] * we
                    return 0
                lax.fori_loop(0, C // 16, _scale, 0)
                pltpu.async_copy(rb[b], acc_sh.at[mb[m].at[0]], ssem[b],
                                 add=True)
        return 0
    lax.fori_loop(0, (n_chunks + 3) // 4, _group, 0)

    _scatter_wait(1)
    plsc.subcore_barrier()

    pltpu.sync_copy(acc_sh.at[pl.ds(s * ROWS_PER_TILE, ROWS_PER_TILE)],
                    out_hbm.at[c, pl.ds(s * ROWS_PER_TILE, ROWS_PER_TILE)])


@functools.cache
def _make_spmm_sc():
    mesh = plsc.VectorSubcoreMesh(core_axis_name="c", subcore_axis_name="s")
    return pl.kernel(
        _spmm_body,
        out_type=jax.ShapeDtypeStruct((NC, NP, D), jnp.float32),
        mesh=mesh,
        scratch_types=[
            pltpu.VMEM((C, D), jnp.float32),
            pltpu.VMEM((C, D), jnp.float32),
            pltpu.VMEM((2, C), jnp.int32),
            pltpu.VMEM((2, C), jnp.int32),
            pltpu.VMEM((2, C), jnp.int32),
            pltpu.VMEM((2, C), jnp.int32),
            pltpu.VMEM((C,), jnp.float32),
            pltpu.VMEM((C,), jnp.float32),
            pltpu.VMEM((C,), jnp.float32),
            pltpu.VMEM((C,), jnp.float32),
            pltpu.VMEM_SHARED((NP, D), jnp.float32),
            pltpu.SemaphoreType.DMA,
            pltpu.SemaphoreType.DMA,
            pltpu.SemaphoreType.DMA,
            pltpu.SemaphoreType.DMA,
            pltpu.SemaphoreType.DMA,
            pltpu.SemaphoreType.DMA,
            pltpu.SemaphoreType.DMA,
            pltpu.SemaphoreType.DMA,
        ],
        name="spmm_sc",
    )


_BLK = 400
_GRID = N_NODES // _BLK


def _mm_body(x_ref, w_ref, o_ref):
    o_ref[...] = jnp.dot(x_ref[...], w_ref[...],
                         preferred_element_type=jnp.float32)


_mm_tc = pl.pallas_call(
    _mm_body,
    grid=(_GRID,),
    in_specs=[
        pl.BlockSpec((_BLK, D), lambda i: (i, 0)),
        pl.BlockSpec((D, D), lambda i: (0, 0)),
    ],
    out_specs=pl.BlockSpec((_BLK, D), lambda i: (i, 0)),
    out_shape=jax.ShapeDtypeStruct((N_NODES, D), jnp.float32),
)


def _fuse_body(p0_ref, p1_ref, b_ref, w_ref, bo_ref, o_ref):
    h = jnp.tanh(p0_ref[...] + p1_ref[...] + b_ref[...])
    o_ref[...] = jnp.dot(h, w_ref[...],
                         preferred_element_type=jnp.float32) + bo_ref[...]


_BLKF = ROWS_PER_TILE
_GRIDF = NP // _BLKF

_fuse_tc = pl.pallas_call(
    _fuse_body,
    grid=(_GRIDF,),
    in_specs=[
        pl.BlockSpec((_BLKF, D), lambda i: (i, 0)),
        pl.BlockSpec((_BLKF, D), lambda i: (i, 0)),
        pl.BlockSpec((1, D), lambda i: (0, 0)),
        pl.BlockSpec((D, D), lambda i: (0, 0)),
        pl.BlockSpec((1, D), lambda i: (0, 0)),
    ],
    out_specs=pl.BlockSpec((_BLKF, D), lambda i: (i, 0)),
    out_shape=jax.ShapeDtypeStruct((NP, D), jnp.float32),
)


def kernel(features, edge, edge_weight, W1, b1, W2, b2, fc_W, fc_b):
    pad = NE_PAD - N_EDGES
    fill = (jnp.arange(pad, dtype=jnp.int32) * 8) % N_NODES
    dst = jnp.concatenate([edge[0].astype(jnp.int32), fill])
    srcx = jnp.concatenate([edge[1].astype(jnp.int32), fill])
    idx = jnp.stack([dst, srcx])
    idx = idx.reshape(2, TOTAL_CHUNKS, C).transpose(1, 0, 2)
    w = jnp.pad(edge_weight, (0, pad)).reshape(TOTAL_CHUNKS, C)
    zeros_row = jnp.zeros((1, D), jnp.float32)

    spmm = _make_spmm_sc()
    xw1 = _mm_tc(features, W1)
    p = spmm(xw1, idx, w)
    xw2 = _fuse_tc(p[0], p[1], b1.reshape(1, D), W2, zeros_row)
    p2 = spmm(xw2, idx, w)
    fcW_pad = jnp.pad(fc_W, ((0, 0), (0, D - fc_W.shape[1])))
    fcb_pad = jnp.pad(fc_b, (0, D - fc_b.shape[0])).reshape(1, D)
    out_full = _fuse_tc(p2[0], p2[1], b2.reshape(1, D), fcW_pad, fcb_pad)
    return out_full[:N_NODES, :fc_W.shape[1]]


# speedup vs baseline: 10.4708x; 1.2711x over previous
import functools

import jax
import jax.numpy as jnp
from jax import lax
from jax.experimental import pallas as pl
from jax.experimental.pallas import tpu as pltpu
from jax.experimental.pallas import tpu_sc as plsc

N_NODES = 10000
N_EDGES = 320000
D = 128

NC = 2
NS = 16
NW = NC * NS
C = 128
K0 = 78
K1 = 80
TOTAL_CHUNKS = NS * (K0 + K1)
NE_PAD = TOTAL_CHUNKS * C
ROWS_PER_TILE = 632
NP = ROWS_PER_TILE * NS
ZB = 79

NBUF = 2
MBUF = 4


def _spmm_body(xw_hbm, dst_hbm, src_hbm, w_hbm, out_hbm,
               rb0, rb1, db0, db1, db2, db3, sb0, sb1, sb2, sb3,
               wb0, wb1, wb2, wb3, acc_sh,
               g0, g1, s0, s1, m0, m1, m2, m3):
    rb = [rb0, rb1]
    db = [db0, db1, db2, db3]
    sb = [sb0, sb1, sb2, sb3]
    wb = [wb0, wb1, wb2, wb3]
    gsem = [g0, g1]
    ssem = [s0, s1]
    msem = [m0, m1, m2, m3]
    c = lax.axis_index("c")
    s = lax.axis_index("s")
    n_chunks = jnp.where(c == 0, K0, K1)
    chunk0 = c * NS * K0 + s * n_chunks

    def _zfill(r, _):
        for jj in range(D // 16):
            rb0[r, pl.ds(jj * 16, 16)] = jnp.zeros((16,), jnp.float32)
        return 0
    lax.fori_loop(0, C, _zfill, 0)
    base_r = s * ROWS_PER_TILE
    for j in range(ROWS_PER_TILE // C):
        pltpu.sync_copy(rb0, acc_sh.at[pl.ds(base_r + j * C, C)])
    rem = ROWS_PER_TILE % C
    if rem:
        pltpu.sync_copy(rb0.at[pl.ds(0, rem)],
                        acc_sh.at[pl.ds(base_r + (ROWS_PER_TILE // C) * C, rem)])
    plsc.subcore_barrier()

    def _meta_start(g, m):
        pltpu.async_copy(dst_hbm.at[chunk0 + g], db[m], msem[m])
        pltpu.async_copy(src_hbm.at[chunk0 + g], sb[m], msem[m])
        pltpu.async_copy(w_hbm.at[chunk0 + g], wb[m], msem[m])

    def _meta_wait(m):
        pltpu.make_async_copy(dst_hbm.at[0], db[m], msem[m]).wait()
        pltpu.make_async_copy(src_hbm.at[0], sb[m], msem[m]).wait()
        pltpu.make_async_copy(w_hbm.at[0], wb[m], msem[m]).wait()

    def _gather_wait(b):
        pltpu.make_async_copy(xw_hbm.at[pl.ds(0, C)], rb[b], gsem[b]).wait()

    def _scatter_wait(b):
        pltpu.make_async_copy(rb[b], acc_sh.at[pl.ds(0, C)], ssem[b]).wait()

    _meta_start(0, 0)
    _meta_start(1, 1)
    _meta_wait(0)
    pltpu.async_copy(xw_hbm.at[sb[0]], rb[0], gsem[0])

    def _group(i, _):
        for bb in range(4):
            g = i * 4 + bb
            b = bb % NBUF
            m = bb

            @pl.when(g < n_chunks)
            def _():
                _gather_wait(b)

                @pl.when(g >= 1)
                def _():
                    _scatter_wait(1 - b)

                @pl.when(g + 1 < n_chunks)
                def _():
                    _meta_wait((m + 1) % MBUF)
                    pltpu.async_copy(xw_hbm.at[sb[(m + 1) % MBUF]],
                                     rb[1 - b], gsem[1 - b])

                @pl.when(g + 2 < n_chunks)
                def _():
                    _meta_start(g + 2, (m + 2) % MBUF)

                def _scale(gg, _c):
                    wv = wb[m][pl.ds(gg * 16, 16)]
                    for k in range(16):
                        e = gg * 16 + k
                        we = wv[k]
                        for jj in range(D // 16):
                            sl = pl.ds(jj * 16, 16)
                            rb[b][e, sl] = rb[b][e, sl] * we
                    return 0
                lax.fori_loop(0, C // 16, _scale, 0)
                pltpu.async_copy(rb[b], acc_sh.at[db[m]], ssem[b],
                                 add=True)
        return 0
    lax.fori_loop(0, (n_chunks + 3) // 4, _group, 0)

    _scatter_wait(1)
    plsc.subcore_barrier()

    pltpu.sync_copy(acc_sh.at[pl.ds(s * ROWS_PER_TILE, ROWS_PER_TILE)],
                    out_hbm.at[c, pl.ds(s * ROWS_PER_TILE, ROWS_PER_TILE)])


@functools.cache
def _make_spmm_sc():
    mesh = plsc.VectorSubcoreMesh(core_axis_name="c", subcore_axis_name="s")
    return pl.kernel(
        _spmm_body,
        out_type=jax.ShapeDtypeStruct((NC, NP, D), jnp.float32),
        mesh=mesh,
        scratch_types=[
            pltpu.VMEM((C, D), jnp.float32),
            pltpu.VMEM((C, D), jnp.float32),
            pltpu.VMEM((C,), jnp.int32),
            pltpu.VMEM((C,), jnp.int32),
            pltpu.VMEM((C,), jnp.int32),
            pltpu.VMEM((C,), jnp.int32),
            pltpu.VMEM((C,), jnp.int32),
            pltpu.VMEM((C,), jnp.int32),
            pltpu.VMEM((C,), jnp.int32),
            pltpu.VMEM((C,), jnp.int32),
            pltpu.VMEM((C,), jnp.float32),
            pltpu.VMEM((C,), jnp.float32),
            pltpu.VMEM((C,), jnp.float32),
            pltpu.VMEM((C,), jnp.float32),
            pltpu.VMEM_SHARED((NP, D), jnp.float32),
            pltpu.SemaphoreType.DMA,
            pltpu.SemaphoreType.DMA,
            pltpu.SemaphoreType.DMA,
            pltpu.SemaphoreType.DMA,
            pltpu.SemaphoreType.DMA,
            pltpu.SemaphoreType.DMA,
            pltpu.SemaphoreType.DMA,
            pltpu.SemaphoreType.DMA,
        ],
        name="spmm_sc",
    )


_BLK = 400
_GRID = N_NODES // _BLK


def _mm_body(x_ref, w_ref, o_ref):
    o_ref[...] = jnp.dot(x_ref[...], w_ref[...],
                         preferred_element_type=jnp.float32)


_mm_tc = pl.pallas_call(
    _mm_body,
    grid=(_GRID,),
    in_specs=[
        pl.BlockSpec((_BLK, D), lambda i: (i, 0)),
        pl.BlockSpec((D, D), lambda i: (0, 0)),
    ],
    out_specs=pl.BlockSpec((_BLK, D), lambda i: (i, 0)),
    out_shape=jax.ShapeDtypeStruct((N_NODES, D), jnp.float32),
)


def _fuse_body(p_ref, b_ref, w_ref, bo_ref, o_ref):
    h = jnp.tanh(p_ref[0] + p_ref[1] + b_ref[...])
    o_ref[...] = jnp.dot(h, w_ref[...],
                         preferred_element_type=jnp.float32) + bo_ref[...]


_BLKF = ROWS_PER_TILE
_GRIDF = NP // _BLKF

_fuse_tc = pl.pallas_call(
    _fuse_body,
    grid=(_GRIDF,),
    in_specs=[
        pl.BlockSpec((2, _BLKF, D), lambda i: (0, i, 0)),
        pl.BlockSpec((1, D), lambda i: (0, 0)),
        pl.BlockSpec((D, D), lambda i: (0, 0)),
        pl.BlockSpec((1, D), lambda i: (0, 0)),
    ],
    out_specs=pl.BlockSpec((_BLKF, D), lambda i: (i, 0)),
    out_shape=jax.ShapeDtypeStruct((NP, D), jnp.float32),
)


def kernel(features, edge, edge_weight, W1, b1, W2, b2, fc_W, fc_b):
    pad = NE_PAD - N_EDGES
    fill = (jnp.arange(pad, dtype=jnp.int32) * 8) % N_NODES
    dst = jnp.concatenate([edge[0].astype(jnp.int32), fill]).reshape(TOTAL_CHUNKS, C)
    srcx = jnp.concatenate([edge[1].astype(jnp.int32), fill]).reshape(TOTAL_CHUNKS, C)
    w = jnp.pad(edge_weight, (0, pad)).reshape(TOTAL_CHUNKS, C)
    zeros_row = jnp.zeros((1, D), jnp.float32)

    spmm = _make_spmm_sc()
    xw1 = _mm_tc(features, W1)
    p = spmm(xw1, dst, srcx, w)
    xw2 = _fuse_tc(p, b1.reshape(1, D), W2, zeros_row)
    p2 = spmm(xw2, dst, srcx, w)
    fcW_pad = jnp.pad(fc_W, ((0, 0), (0, D - fc_W.shape[1])))
    fcb_pad = jnp.pad(fc_b, (0, D - fc_b.shape[0])).reshape(1, D)
    out_full = _fuse_tc(p2, b2.reshape(1, D), fcW_pad, fcb_pad)
    return out_full[:N_NODES, :fc_W.shape[1]]


# speedup vs baseline: 10.9711x; 1.0478x over previous
import functools

import jax
import jax.numpy as jnp
from jax import lax
from jax.experimental import pallas as pl
from jax.experimental.pallas import tpu as pltpu
from jax.experimental.pallas import tpu_sc as plsc

N_NODES = 10000
N_EDGES = 320000
D = 128

NC = 2
NS = 16
NW = NC * NS
C = 128
K0 = 78
K1 = 80
TOTAL_CHUNKS = NS * (K0 + K1)
NE_PAD = TOTAL_CHUNKS * C
ROWS_PER_TILE = 632
NP = ROWS_PER_TILE * NS
ZB = 79

NBUF = 2
MBUF = 4


def _spmm_body(xw_hbm, dst_hbm, src_hbm, w_hbm, out_hbm,
               rb0, rb1, db0, db1, db2, db3, sb0, sb1, sb2, sb3,
               wb0, wb1, wb2, wb3, acc_sh,
               g0, g1, s0, s1, m0, m1, m2, m3):
    rb = [rb0, rb1]
    db = [db0, db1, db2, db3]
    sb = [sb0, sb1, sb2, sb3]
    wb = [wb0, wb1, wb2, wb3]
    gsem = [g0, g1]
    ssem = [s0, s1]
    msem = [m0, m1, m2, m3]
    c = lax.axis_index("c")
    s = lax.axis_index("s")
    n_chunks = jnp.where(c == 0, K0, K1)
    chunk0 = c * NS * K0 + s * n_chunks

    def _zfill(r, _):
        for jj in range(D // 16):
            rb0[r, pl.ds(jj * 16, 16)] = jnp.zeros((16,), jnp.float32)
        return 0
    lax.fori_loop(0, C, _zfill, 0)
    base_r = s * ROWS_PER_TILE
    for j in range(ROWS_PER_TILE // C):
        pltpu.sync_copy(rb0, acc_sh.at[pl.ds(base_r + j * C, C)])
    rem = ROWS_PER_TILE % C
    if rem:
        pltpu.sync_copy(rb0.at[pl.ds(0, rem)],
                        acc_sh.at[pl.ds(base_r + (ROWS_PER_TILE // C) * C, rem)])
    plsc.subcore_barrier()

    def _meta_start(g, m):
        pltpu.async_copy(dst_hbm.at[chunk0 + g], db[m], msem[m])
        pltpu.async_copy(src_hbm.at[chunk0 + g], sb[m], msem[m])
        pltpu.async_copy(w_hbm.at[chunk0 + g], wb[m], msem[m])

    def _meta_wait(m):
        pltpu.make_async_copy(dst_hbm.at[0], db[m], msem[m]).wait()
        pltpu.make_async_copy(src_hbm.at[0], sb[m], msem[m]).wait()
        pltpu.make_async_copy(w_hbm.at[0], wb[m], msem[m]).wait()

    def _gather_wait(b):
        pltpu.make_async_copy(xw_hbm.at[pl.ds(0, C)], rb[b], gsem[b]).wait()

    def _scatter_wait(b):
        pltpu.make_async_copy(rb[b], acc_sh.at[pl.ds(0, C)], ssem[b]).wait()

    _meta_start(0, 0)
    _meta_start(1, 1)
    _meta_wait(0)
    pltpu.async_copy(xw_hbm.at[sb[0]], rb[0], gsem[0])

    def _group(i, _):
        for bb in range(4):
            g = i * 4 + bb
            b = bb % NBUF
            m = bb

            @pl.when(g < n_chunks)
            def _():
                _gather_wait(b)

                @pl.when(g >= 1)
                def _():
                    _scatter_wait(1 - b)

                @pl.when(g + 1 < n_chunks)
                def _():
                    _meta_wait((m + 1) % MBUF)
                    pltpu.async_copy(xw_hbm.at[sb[(m + 1) % MBUF]],
                                     rb[1 - b], gsem[1 - b])

                @pl.when(g + 2 < n_chunks)
                def _():
                    _meta_start(g + 2, (m + 2) % MBUF)

                def _scale(gg, _c):
                    wv = wb[m][pl.ds(gg * 16, 16)]
                    for k in range(16):
                        e = gg * 16 + k
                        we = wv[k]
                        for jj in range(D // 16):
                            sl = pl.ds(jj * 16, 16)
                            rb[b][e, sl] = rb[b][e, sl] * we
                    return 0
                lax.fori_loop(0, C // 16, _scale, 0)
                pltpu.async_copy(rb[b], acc_sh.at[db[m]], ssem[b],
                                 add=True)
        return 0
    lax.fori_loop(0, (n_chunks + 3) // 4, _group, 0)

    _scatter_wait(1)
    plsc.subcore_barrier()

    pltpu.sync_copy(acc_sh.at[pl.ds(s * ROWS_PER_TILE, ROWS_PER_TILE)],
                    out_hbm.at[c, pl.ds(s * ROWS_PER_TILE, ROWS_PER_TILE)])


@functools.cache
def _make_spmm_sc():
    mesh = plsc.VectorSubcoreMesh(core_axis_name="c", subcore_axis_name="s")
    return pl.kernel(
        _spmm_body,
        out_type=jax.ShapeDtypeStruct((NC, NP, D), jnp.float32),
        mesh=mesh,
        scratch_types=[
            pltpu.VMEM((C, D), jnp.float32),
            pltpu.VMEM((C, D), jnp.float32),
            pltpu.VMEM((C,), jnp.int32),
            pltpu.VMEM((C,), jnp.int32),
            pltpu.VMEM((C,), jnp.int32),
            pltpu.VMEM((C,), jnp.int32),
            pltpu.VMEM((C,), jnp.int32),
            pltpu.VMEM((C,), jnp.int32),
            pltpu.VMEM((C,), jnp.int32),
            pltpu.VMEM((C,), jnp.int32),
            pltpu.VMEM((C,), jnp.float32),
            pltpu.VMEM((C,), jnp.float32),
            pltpu.VMEM((C,), jnp.float32),
            pltpu.VMEM((C,), jnp.float32),
            pltpu.VMEM_SHARED((NP, D), jnp.float32),
            pltpu.SemaphoreType.DMA,
            pltpu.SemaphoreType.DMA,
            pltpu.SemaphoreType.DMA,
            pltpu.SemaphoreType.DMA,
            pltpu.SemaphoreType.DMA,
            pltpu.SemaphoreType.DMA,
            pltpu.SemaphoreType.DMA,
            pltpu.SemaphoreType.DMA,
        ],
        name="spmm_sc",
    )


_BLKF = ROWS_PER_TILE
_GRIDF = NP // _BLKF


def _fuse_a_body(p_ref, w_ref, b_ref, o_ref):
    s = p_ref[0] + p_ref[1]
    o_ref[...] = jnp.tanh(
        jnp.dot(s, w_ref[...], preferred_element_type=jnp.float32)
        + b_ref[...])


_fuse_a_tc = pl.pallas_call(
    _fuse_a_body,
    grid=(_GRIDF,),
    in_specs=[
        pl.BlockSpec((2, _BLKF, D), lambda i: (0, i, 0)),
        pl.BlockSpec((D, D), lambda i: (0, 0)),
        pl.BlockSpec((1, D), lambda i: (0, 0)),
    ],
    out_specs=pl.BlockSpec((_BLKF, D), lambda i: (i, 0)),
    out_shape=jax.ShapeDtypeStruct((NP, D), jnp.float32),
)


def _fuse_b_body(p_ref, w_ref, b_ref, wf_ref, bf_ref, o_ref):
    s = p_ref[0] + p_ref[1]
    h = jnp.tanh(
        jnp.dot(s, w_ref[...], preferred_element_type=jnp.float32)
        + b_ref[...])
    o_ref[...] = jnp.dot(h, wf_ref[...],
                         preferred_element_type=jnp.float32) + bf_ref[...]


_fuse_b_tc = pl.pallas_call(
    _fuse_b_body,
    grid=(_GRIDF,),
    in_specs=[
        pl.BlockSpec((2, _BLKF, D), lambda i: (0, i, 0)),
        pl.BlockSpec((D, D), lambda i: (0, 0)),
        pl.BlockSpec((1, D), lambda i: (0, 0)),
        pl.BlockSpec((D, D), lambda i: (0, 0)),
        pl.BlockSpec((1, D), lambda i: (0, 0)),
    ],
    out_specs=pl.BlockSpec((_BLKF, D), lambda i: (i, 0)),
    out_shape=jax.ShapeDtypeStruct((NP, D), jnp.float32),
)


def kernel(features, edge, edge_weight, W1, b1, W2, b2, fc_W, fc_b):
    pad = NE_PAD - N_EDGES
    fill = (jnp.arange(pad, dtype=jnp.int32) * 8) % N_NODES
    dst = jnp.concatenate([edge[0].astype(jnp.int32), fill]).reshape(TOTAL_CHUNKS, C)
    srcx = jnp.concatenate([edge[1].astype(jnp.int32), fill]).reshape(TOTAL_CHUNKS, C)
    w = jnp.pad(edge_weight, (0, pad)).reshape(TOTAL_CHUNKS, C)

    spmm = _make_spmm_sc()
    p = spmm(features, dst, srcx, w)
    h1 = _fuse_a_tc(p, W1, b1.reshape(1, D))
    p2 = spmm(h1, dst, srcx, w)
    fcW_pad = jnp.pad(fc_W, ((0, 0), (0, D - fc_W.shape[1])))
    fcb_pad = jnp.pad(fc_b, (0, D - fc_b.shape[0])).reshape(1, D)
    out_full = _fuse_b_tc(p2, W2, b2.reshape(1, D), fcW_pad, fcb_pad)
    return out_full[:N_NODES, :fc_W.shape[1]]
